# baseline (speedup 1.0000x reference)
import jax
import jax.numpy as jnp
from jax.experimental import pallas as pl
from jax.experimental.pallas import tpu as pltpu

_H = 128
_K = 32
_BLOCK = 625


def _cell_kernel(nh_ref, nc_ref, fin_ref, iou_ref, uf_ref, h_ref, c_ref):
    b = fin_ref.shape[1]
    fg = jax.lax.dot_general(
        nh_ref[...], uf_ref[...], (((1,), (1,)), ((), ())),
        preferred_element_type=jnp.float32,
    )
    f = jax.nn.sigmoid(fg.reshape(b, _K, _H) + fin_ref[0][:, None, :])
    c_aggr = jnp.sum(f * nc_ref[...].reshape(b, _K, _H), axis=1)
    iou = iou_ref[0]
    i = jax.nn.sigmoid(iou[:, :_H])
    o = jax.nn.sigmoid(iou[:, _H:2 * _H])
    u = jnp.tanh(iou[:, 2 * _H:])
    c = i * u + c_aggr
    h_ref[0] = o * jnp.tanh(c)
    c_ref[0] = c


def kernel(neighbour_h, neighbour_c, f_input, iou_input, U_f):
    n, k, h = neighbour_h.shape
    b = _BLOCK
    g = n // b
    nh2 = neighbour_h.reshape(n * k, h)
    nc2 = neighbour_c.reshape(n * k, h)
    fin3 = f_input.reshape(g, b, h)
    iou3 = iou_input.reshape(g, b, 3 * h)
    h_out, c_out = pl.pallas_call(
        _cell_kernel,
        grid=(g,),
        in_specs=[
            pl.BlockSpec((b * k, h), lambda i: (i, 0)),
            pl.BlockSpec((b * k, h), lambda i: (i, 0)),
            pl.BlockSpec((1, b, h), lambda i: (i, 0, 0)),
            pl.BlockSpec((1, b, 3 * h), lambda i: (i, 0, 0)),
            pl.BlockSpec((h, h), lambda i: (0, 0)),
        ],
        out_specs=(
            pl.BlockSpec((1, b, h), lambda i: (i, 0, 0)),
            pl.BlockSpec((1, b, h), lambda i: (i, 0, 0)),
        ),
        out_shape=(
            jax.ShapeDtypeStruct((g, b, h), jnp.float32),
            jax.ShapeDtypeStruct((g, b, h), jnp.float32),
        ),
        compiler_params=pltpu.CompilerParams(
            dimension_semantics=("parallel",),
        ),
    )(nh2, nc2, fin3, iou3, U_f)
    return h_out.reshape(n, h), c_out.reshape(n, h)


# speedup vs baseline: 1.2617x; 1.2617x over previous
import jax
import jax.numpy as jnp
from jax.experimental import pallas as pl
from jax.experimental.pallas import tpu as pltpu

_H = 128
_K = 32
_BLOCK = 400


def _cell_kernel(nh_ref, nc_ref, fin_ref, iou_ref, uf_ref, h_ref, c_ref):
    b = nh_ref.shape[0]
    nh = nh_ref[...].reshape(b * _K, _H)
    fg = jax.lax.dot_general(
        nh, uf_ref[...], (((1,), (1,)), ((), ())),
        preferred_element_type=jnp.float32,
    )
    f = jax.nn.sigmoid(fg.reshape(b, _K, _H) + fin_ref[...][:, None, :])
    c_aggr = jnp.sum(f * nc_ref[...], axis=1)
    iou = iou_ref[...]
    i = jax.nn.sigmoid(iou[:, :_H])
    o = jax.nn.sigmoid(iou[:, _H:2 * _H])
    u = jnp.tanh(iou[:, 2 * _H:])
    c = i * u + c_aggr
    h_ref[...] = o * jnp.tanh(c)
    c_ref[...] = c


def kernel(neighbour_h, neighbour_c, f_input, iou_input, U_f):
    n, k, h = neighbour_h.shape
    b = _BLOCK
    return pl.pallas_call(
        _cell_kernel,
        grid=(n // b,),
        in_specs=[
            pl.BlockSpec((b, k, h), lambda i: (i, 0, 0)),
            pl.BlockSpec((b, k, h), lambda i: (i, 0, 0)),
            pl.BlockSpec((b, h), lambda i: (i, 0)),
            pl.BlockSpec((b, 3 * h), lambda i: (i, 0)),
            pl.BlockSpec((h, h), lambda i: (0, 0)),
        ],
        out_specs=(
            pl.BlockSpec((b, h), lambda i: (i, 0)),
            pl.BlockSpec((b, h), lambda i: (i, 0)),
        ),
        out_shape=(
            jax.ShapeDtypeStruct((n, h), jnp.float32),
            jax.ShapeDtypeStruct((n, h), jnp.float32),
        ),
        compiler_params=pltpu.CompilerParams(
            dimension_semantics=("parallel",),
        ),
    )(neighbour_h, neighbour_c, f_input, iou_input, U_f)


# speedup vs baseline: 1.2632x; 1.0012x over previous
import jax
import jax.numpy as jnp
from jax.experimental import pallas as pl
from jax.experimental.pallas import tpu as pltpu

_H = 128
_K = 32
_BLOCK = 400


def _cell_kernel(nh_ref, nc_ref, fin_ref, iou_ref, uf_ref, h_ref, c_ref):
    b = fin_ref.shape[0]
    fg = jax.lax.dot_general(
        nh_ref[...], uf_ref[...], (((1,), (1,)), ((), ())),
        preferred_element_type=jnp.float32,
    )
    f = jax.nn.sigmoid(fg.reshape(b, _K, _H) + fin_ref[...][:, None, :])
    c_aggr = jnp.sum(f * nc_ref[...].reshape(b, _K, _H), axis=1)
    iou = iou_ref[...]
    i = jax.nn.sigmoid(iou[:, :_H])
    o = jax.nn.sigmoid(iou[:, _H:2 * _H])
    u = jnp.tanh(iou[:, 2 * _H:])
    c = i * u + c_aggr
    h_ref[...] = o * jnp.tanh(c)
    c_ref[...] = c


def kernel(neighbour_h, neighbour_c, f_input, iou_input, U_f):
    n, k, h = neighbour_h.shape
    b = _BLOCK
    nh2 = neighbour_h.reshape(n * k, h)
    nc2 = neighbour_c.reshape(n * k, h)
    return pl.pallas_call(
        _cell_kernel,
        grid=(n // b,),
        in_specs=[
            pl.BlockSpec((b * k, h), lambda i: (i, 0)),
            pl.BlockSpec((b * k, h), lambda i: (i, 0)),
            pl.BlockSpec((b, h), lambda i: (i, 0)),
            pl.BlockSpec((b, 3 * h), lambda i: (i, 0)),
            pl.BlockSpec((h, h), lambda i: (0, 0)),
        ],
        out_specs=(
            pl.BlockSpec((b, h), lambda i: (i, 0)),
            pl.BlockSpec((b, h), lambda i: (i, 0)),
        ),
        out_shape=(
            jax.ShapeDtypeStruct((n, h), jnp.float32),
            jax.ShapeDtypeStruct((n, h), jnp.float32),
        ),
        compiler_params=pltpu.CompilerParams(
            dimension_semantics=("parallel",),
        ),
    )(nh2, nc2, f_input, iou_input, U_f)


# speedup vs baseline: 1.2677x; 1.0036x over previous
import jax
import jax.numpy as jnp
from jax.experimental import pallas as pl
from jax.experimental.pallas import tpu as pltpu

_H = 128
_K = 32
_BLOCK = 400


def _cell_kernel(nh_ref, nc_ref, fin_ref, iou_ref, uf_ref, h_ref, c_ref):
    b = fin_ref.shape[0]
    fg = jax.lax.dot_general(
        nh_ref[...], uf_ref[...], (((1,), (1,)), ((), ())),
        preferred_element_type=jnp.float32,
    )
    fa = fg.reshape(b, _K, _H) + fin_ref[...][:, None, :]
    f = 0.5 * jnp.tanh(0.5 * fa) + 0.5
    c_aggr = jnp.sum(f * nc_ref[...].reshape(b, _K, _H), axis=1)
    iou = iou_ref[...]
    i = jax.nn.sigmoid(iou[:, :_H])
    o = jax.nn.sigmoid(iou[:, _H:2 * _H])
    u = jnp.tanh(iou[:, 2 * _H:])
    c = i * u + c_aggr
    h_ref[...] = o * jnp.tanh(c)
    c_ref[...] = c


def kernel(neighbour_h, neighbour_c, f_input, iou_input, U_f):
    n, k, h = neighbour_h.shape
    b = _BLOCK
    nh2 = neighbour_h.reshape(n * k, h)
    nc2 = neighbour_c.reshape(n * k, h)
    return pl.pallas_call(
        _cell_kernel,
        grid=(n // b,),
        in_specs=[
            pl.BlockSpec((b * k, h), lambda i: (i, 0)),
            pl.BlockSpec((b * k, h), lambda i: (i, 0)),
            pl.BlockSpec((b, h), lambda i: (i, 0)),
            pl.BlockSpec((b, 3 * h), lambda i: (i, 0)),
            pl.BlockSpec((h, h), lambda i: (0, 0)),
        ],
        out_specs=(
            pl.BlockSpec((b, h), lambda i: (i, 0)),
            pl.BlockSpec((b, h), lambda i: (i, 0)),
        ),
        out_shape=(
            jax.ShapeDtypeStruct((n, h), jnp.float32),
            jax.ShapeDtypeStruct((n, h), jnp.float32),
        ),
        compiler_params=pltpu.CompilerParams(
            dimension_semantics=("parallel",),
        ),
    )(nh2, nc2, f_input, iou_input, U_f)
